# baseline (device time: 45853 ns/iter reference)
import jax
import jax.numpy as jnp
from jax import lax
from jax.experimental import pallas as pl
from jax.experimental.pallas import tpu as pltpu

N_DEV = 8
F8 = jnp.float8_e4m3fn


def kernel(x, w_mat):
    m_per, k = x.shape
    _, n = w_mat.shape
    n_per = n // N_DEV

    def body(x_ref, w_ref, out_ref, y_ref, qs_ref, qr_ref, gmax_ref,
             amax_send_sems, amax_recv_sems, send_sems, recv_sems):
        me = lax.axis_index("i")

        barrier = pltpu.get_barrier_semaphore()
        for d in range(N_DEV):
            @pl.when(me != d)
            def _():
                pl.semaphore_signal(
                    barrier, inc=1,
                    device_id=(d,), device_id_type=pl.DeviceIdType.MESH,
                )
        pl.semaphore_wait(barrier, N_DEV - 1)

        y_ref[...] = jnp.maximum(
            jnp.dot(x_ref[...], w_ref[...], preferred_element_type=jnp.float32),
            0.0,
        )

        local_amax = jnp.max(y_ref[...])
        gmax_ref[pl.ds(me, 1), :] = jnp.full((1, 128), local_amax, jnp.float32)

        for o in range(1, N_DEV):
            dst = (me + o) % N_DEV
            rdma = pltpu.make_async_remote_copy(
                src_ref=gmax_ref.at[pl.ds(me, 1), :],
                dst_ref=gmax_ref.at[pl.ds(me, 1), :],
                send_sem=amax_send_sems.at[dst],
                recv_sem=amax_recv_sems.at[me],
                device_id=(dst,),
                device_id_type=pl.DeviceIdType.MESH,
            )
            rdma.start()
        for s in range(N_DEV):
            @pl.when(me != s)
            def _():
                recv = pltpu.make_async_remote_copy(
                    src_ref=gmax_ref.at[pl.ds(s, 1), :],
                    dst_ref=gmax_ref.at[pl.ds(s, 1), :],
                    send_sem=amax_send_sems.at[s],
                    recv_sem=amax_recv_sems.at[s],
                    device_id=(s,),
                    device_id_type=pl.DeviceIdType.MESH,
                )
                recv.wait_recv()

        gmax = jnp.max(gmax_ref[...])
        scale = gmax / 448.0
        inv_scale = 1.0 / scale

        for j in range(N_DEV):
            qs_ref[j, :, :] = (
                y_ref[:, j * n_per:(j + 1) * n_per] * inv_scale
            ).astype(F8)

        for o in range(1, N_DEV):
            dst = (me + o) % N_DEV
            rdma = pltpu.make_async_remote_copy(
                src_ref=qs_ref.at[dst],
                dst_ref=qr_ref.at[me],
                send_sem=send_sems.at[dst],
                recv_sem=recv_sems.at[me],
                device_id=(dst,),
                device_id_type=pl.DeviceIdType.MESH,
            )
            rdma.start()

        own = qs_ref[pl.ds(me, 1)]
        out_ref[pl.ds(me * m_per, m_per), :] = own[0].astype(jnp.float32) * scale

        for s in range(N_DEV):
            @pl.when(me != s)
            def _():
                recv = pltpu.make_async_remote_copy(
                    src_ref=qs_ref.at[s],
                    dst_ref=qr_ref.at[s],
                    send_sem=send_sems.at[s],
                    recv_sem=recv_sems.at[s],
                    device_id=(s,),
                    device_id_type=pl.DeviceIdType.MESH,
                )
                recv.wait_recv()
                out_ref[s * m_per:(s + 1) * m_per, :] = (
                    qr_ref[s].astype(jnp.float32) * scale
                )

        for o in range(1, N_DEV):
            dst = (me + o) % N_DEV
            snd = pltpu.make_async_remote_copy(
                src_ref=qs_ref.at[dst],
                dst_ref=qr_ref.at[me],
                send_sem=send_sems.at[dst],
                recv_sem=recv_sems.at[me],
                device_id=(dst,),
                device_id_type=pl.DeviceIdType.MESH,
            )
            snd.wait_send()
            asnd = pltpu.make_async_remote_copy(
                src_ref=gmax_ref.at[pl.ds(me, 1), :],
                dst_ref=gmax_ref.at[pl.ds(me, 1), :],
                send_sem=amax_send_sems.at[dst],
                recv_sem=amax_recv_sems.at[me],
                device_id=(dst,),
                device_id_type=pl.DeviceIdType.MESH,
            )
            asnd.wait_send()

    out_shape = jax.ShapeDtypeStruct((N_DEV * m_per, n_per), jnp.float32)
    return pl.pallas_call(
        body,
        out_shape=out_shape,
        in_specs=[
            pl.BlockSpec(memory_space=pltpu.VMEM),
            pl.BlockSpec(memory_space=pltpu.VMEM),
        ],
        out_specs=pl.BlockSpec(memory_space=pltpu.VMEM),
        scratch_shapes=[
            pltpu.VMEM((m_per, n), jnp.float32),
            pltpu.VMEM((N_DEV, m_per, n_per), F8),
            pltpu.VMEM((N_DEV, m_per, n_per), F8),
            pltpu.VMEM((N_DEV, 128), jnp.float32),
            pltpu.SemaphoreType.DMA((N_DEV,)),
            pltpu.SemaphoreType.DMA((N_DEV,)),
            pltpu.SemaphoreType.DMA((N_DEV,)),
            pltpu.SemaphoreType.DMA((N_DEV,)),
        ],
        compiler_params=pltpu.CompilerParams(
            collective_id=0,
            vmem_limit_bytes=100 * 1024 * 1024,
        ),
    )(x, w_mat)


# device time: 44708 ns/iter; 1.0256x vs baseline; 1.0256x over previous
import jax
import jax.numpy as jnp
from jax import lax
from jax.experimental import pallas as pl
from jax.experimental.pallas import tpu as pltpu

N_DEV = 8
F8 = jnp.float8_e4m3fn


def kernel(x, w_mat):
    m_per, k = x.shape
    _, n = w_mat.shape
    n_per = n // N_DEV

    def body(x_hbm, w_hbm, out_ref, x_ref, wbuf_ref, y_ref, qs_ref, qr_ref,
             gmax_ref, x_sem, w_sems,
             amax_send_sems, amax_recv_sems, send_sems, recv_sems):
        me = lax.axis_index("i")

        x_copy = pltpu.make_async_copy(x_hbm, x_ref, x_sem)
        x_copy.start()

        def w_copy(j):
            return pltpu.make_async_copy(
                w_hbm.at[:, pl.ds(j * n_per, n_per)],
                wbuf_ref.at[j % 2],
                w_sems.at[j % 2],
            )
        w_copy(0).start()

        barrier = pltpu.get_barrier_semaphore()
        for d in range(N_DEV):
            @pl.when(me != d)
            def _():
                pl.semaphore_signal(
                    barrier, inc=1,
                    device_id=(d,), device_id_type=pl.DeviceIdType.MESH,
                )
        pl.semaphore_wait(barrier, N_DEV - 1)

        local_amax = jnp.float32(0.0)
        for j in range(N_DEV):
            if j + 1 < N_DEV:
                w_copy(j + 1).start()
            w_copy(j).wait()
            if j == 0:
                x_copy.wait()
            blk = jnp.maximum(
                jnp.dot(x_ref[...], wbuf_ref[j % 2],
                        preferred_element_type=jnp.float32),
                0.0,
            )
            y_ref[:, j * n_per:(j + 1) * n_per] = blk
            local_amax = jnp.maximum(local_amax, jnp.max(blk))

        gmax_ref[pl.ds(me, 1), :] = jnp.full((1, 128), local_amax, jnp.float32)
        for o in range(1, N_DEV):
            dst = (me + o) % N_DEV
            rdma = pltpu.make_async_remote_copy(
                src_ref=gmax_ref.at[pl.ds(me, 1), :],
                dst_ref=gmax_ref.at[pl.ds(me, 1), :],
                send_sem=amax_send_sems.at[dst],
                recv_sem=amax_recv_sems.at[me],
                device_id=(dst,),
                device_id_type=pl.DeviceIdType.MESH,
            )
            rdma.start()
        for s in range(N_DEV):
            @pl.when(me != s)
            def _():
                recv = pltpu.make_async_remote_copy(
                    src_ref=gmax_ref.at[pl.ds(s, 1), :],
                    dst_ref=gmax_ref.at[pl.ds(s, 1), :],
                    send_sem=amax_send_sems.at[s],
                    recv_sem=amax_recv_sems.at[s],
                    device_id=(s,),
                    device_id_type=pl.DeviceIdType.MESH,
                )
                recv.wait_recv()

        gmax = jnp.max(gmax_ref[...])
        scale = gmax / 448.0
        inv_scale = 1.0 / scale

        for j in range(N_DEV):
            qs_ref[j, :, :] = (
                y_ref[:, j * n_per:(j + 1) * n_per] * inv_scale
            ).astype(F8)
            @pl.when(me != j)
            def _(j=j):
                rdma = pltpu.make_async_remote_copy(
                    src_ref=qs_ref.at[j],
                    dst_ref=qr_ref.at[me],
                    send_sem=send_sems.at[j],
                    recv_sem=recv_sems.at[me],
                    device_id=(j,),
                    device_id_type=pl.DeviceIdType.MESH,
                )
                rdma.start()

        own = qs_ref[pl.ds(me, 1)]
        out_ref[pl.ds(me * m_per, m_per), :] = own[0].astype(jnp.float32) * scale

        for s in range(N_DEV):
            @pl.when(me != s)
            def _(s=s):
                recv = pltpu.make_async_remote_copy(
                    src_ref=qs_ref.at[s],
                    dst_ref=qr_ref.at[s],
                    send_sem=send_sems.at[s],
                    recv_sem=recv_sems.at[s],
                    device_id=(s,),
                    device_id_type=pl.DeviceIdType.MESH,
                )
                recv.wait_recv()
                out_ref[s * m_per:(s + 1) * m_per, :] = (
                    qr_ref[s].astype(jnp.float32) * scale
                )

        for o in range(1, N_DEV):
            dst = (me + o) % N_DEV
            snd = pltpu.make_async_remote_copy(
                src_ref=qs_ref.at[dst],
                dst_ref=qr_ref.at[me],
                send_sem=send_sems.at[dst],
                recv_sem=recv_sems.at[me],
                device_id=(dst,),
                device_id_type=pl.DeviceIdType.MESH,
            )
            snd.wait_send()
            asnd = pltpu.make_async_remote_copy(
                src_ref=gmax_ref.at[pl.ds(me, 1), :],
                dst_ref=gmax_ref.at[pl.ds(me, 1), :],
                send_sem=amax_send_sems.at[dst],
                recv_sem=amax_recv_sems.at[me],
                device_id=(dst,),
                device_id_type=pl.DeviceIdType.MESH,
            )
            asnd.wait_send()

    out_shape = jax.ShapeDtypeStruct((N_DEV * m_per, n_per), jnp.float32)
    return pl.pallas_call(
        body,
        out_shape=out_shape,
        in_specs=[
            pl.BlockSpec(memory_space=pl.ANY),
            pl.BlockSpec(memory_space=pl.ANY),
        ],
        out_specs=pl.BlockSpec(memory_space=pltpu.VMEM),
        scratch_shapes=[
            pltpu.VMEM((m_per, k), jnp.float32),
            pltpu.VMEM((2, k, n_per), jnp.float32),
            pltpu.VMEM((m_per, n), jnp.float32),
            pltpu.VMEM((N_DEV, m_per, n_per), F8),
            pltpu.VMEM((N_DEV, m_per, n_per), F8),
            pltpu.VMEM((N_DEV, 128), jnp.float32),
            pltpu.SemaphoreType.DMA,
            pltpu.SemaphoreType.DMA((2,)),
            pltpu.SemaphoreType.DMA((N_DEV,)),
            pltpu.SemaphoreType.DMA((N_DEV,)),
            pltpu.SemaphoreType.DMA((N_DEV,)),
            pltpu.SemaphoreType.DMA((N_DEV,)),
        ],
        compiler_params=pltpu.CompilerParams(
            collective_id=0,
            vmem_limit_bytes=100 * 1024 * 1024,
        ),
    )(x, w_mat)


# device time: 44677 ns/iter; 1.0263x vs baseline; 1.0007x over previous
import jax
import jax.numpy as jnp
from jax import lax
from jax.experimental import pallas as pl
from jax.experimental.pallas import tpu as pltpu

N_DEV = 8
F8 = jnp.float8_e4m3fn


def kernel(x, w_mat):
    m_per, k = x.shape
    _, n = w_mat.shape
    n_per = n // N_DEV

    def body(x_hbm, w_hbm, out_ref, *s):
        x_ref, wb0, wb1, y_ref = s[0], s[1], s[2], s[3]
        qs = list(s[4:4 + N_DEV])
        qr_ref, gmax_ref = s[12], s[13]
        x_sem, w_sems = s[14], s[15]
        amax_send_sems, amax_recv_sems = s[16], s[17]
        send_sems, recv_sems = s[18], s[19]
        wbufs = (wb0, wb1)

        me = lax.axis_index("i")

        x_copy = pltpu.make_async_copy(x_hbm, x_ref, x_sem)
        x_copy.start()

        def w_copy(j):
            return pltpu.make_async_copy(
                w_hbm.at[:, pl.ds(j * n_per, n_per)],
                wbufs[j % 2],
                w_sems.at[j % 2],
            )
        w_copy(0).start()

        barrier = pltpu.get_barrier_semaphore()
        for d in range(N_DEV):
            @pl.when(me != d)
            def _():
                pl.semaphore_signal(
                    barrier, inc=1,
                    device_id=(d,), device_id_type=pl.DeviceIdType.MESH,
                )
        pl.semaphore_wait(barrier, N_DEV - 1)

        local_amax = jnp.float32(0.0)
        for j in range(N_DEV):
            if j + 1 < N_DEV:
                w_copy(j + 1).start()
            w_copy(j).wait()
            if j == 0:
                x_copy.wait()
            blk = jnp.maximum(
                jnp.dot(x_ref[...], wbufs[j % 2][...],
                        preferred_element_type=jnp.float32),
                0.0,
            )
            y_ref[:, j * n_per:(j + 1) * n_per] = blk
            local_amax = jnp.maximum(local_amax, jnp.max(blk))

        gmax_ref[pl.ds(me, 1), :] = jnp.full((1, 128), local_amax, jnp.float32)
        for o in range(1, N_DEV):
            dst = (me + o) % N_DEV
            rdma = pltpu.make_async_remote_copy(
                src_ref=gmax_ref.at[pl.ds(me, 1), :],
                dst_ref=gmax_ref.at[pl.ds(me, 1), :],
                send_sem=amax_send_sems.at[dst],
                recv_sem=amax_recv_sems.at[me],
                device_id=(dst,),
                device_id_type=pl.DeviceIdType.MESH,
            )
            rdma.start()
        for t in range(N_DEV):
            @pl.when(me != t)
            def _(t=t):
                recv = pltpu.make_async_remote_copy(
                    src_ref=gmax_ref.at[pl.ds(t, 1), :],
                    dst_ref=gmax_ref.at[pl.ds(t, 1), :],
                    send_sem=amax_send_sems.at[t],
                    recv_sem=amax_recv_sems.at[t],
                    device_id=(t,),
                    device_id_type=pl.DeviceIdType.MESH,
                )
                recv.wait_recv()

        gmax = jnp.max(gmax_ref[...])
        scale = gmax / 448.0
        inv_scale = 1.0 / scale

        for j in range(N_DEV):
            qs[j][...] = (
                y_ref[:, j * n_per:(j + 1) * n_per] * inv_scale
            ).astype(F8)
            @pl.when(me != j)
            def _(j=j):
                rdma = pltpu.make_async_remote_copy(
                    src_ref=qs[j],
                    dst_ref=qr_ref.at[me],
                    send_sem=send_sems.at[j],
                    recv_sem=recv_sems.at[me],
                    device_id=(j,),
                    device_id_type=pl.DeviceIdType.MESH,
                )
                rdma.start()

        own_q = (y_ref[:, pl.ds(me * n_per, n_per)] * inv_scale).astype(F8)
        out_ref[pl.ds(me * m_per, m_per), :] = own_q.astype(jnp.float32) * scale

        for t in range(N_DEV):
            @pl.when(me != t)
            def _(t=t):
                recv = pltpu.make_async_remote_copy(
                    src_ref=qs[t],
                    dst_ref=qr_ref.at[t],
                    send_sem=send_sems.at[t],
                    recv_sem=recv_sems.at[t],
                    device_id=(t,),
                    device_id_type=pl.DeviceIdType.MESH,
                )
                recv.wait_recv()
                out_ref[t * m_per:(t + 1) * m_per, :] = (
                    qr_ref[t].astype(jnp.float32) * scale
                )

        for j in range(N_DEV):
            @pl.when(me != j)
            def _(j=j):
                pltpu.make_async_remote_copy(
                    src_ref=qs[j],
                    dst_ref=qr_ref.at[j],
                    send_sem=send_sems.at[j],
                    recv_sem=recv_sems.at[j],
                    device_id=(j,),
                    device_id_type=pl.DeviceIdType.MESH,
                ).wait_send()
                pltpu.make_async_remote_copy(
                    src_ref=gmax_ref.at[pl.ds(j, 1), :],
                    dst_ref=gmax_ref.at[pl.ds(j, 1), :],
                    send_sem=amax_send_sems.at[j],
                    recv_sem=amax_recv_sems.at[j],
                    device_id=(j,),
                    device_id_type=pl.DeviceIdType.MESH,
                ).wait_send()

    out_shape = jax.ShapeDtypeStruct((N_DEV * m_per, n_per), jnp.float32)
    return pl.pallas_call(
        body,
        out_shape=out_shape,
        in_specs=[
            pl.BlockSpec(memory_space=pl.ANY),
            pl.BlockSpec(memory_space=pl.ANY),
        ],
        out_specs=pl.BlockSpec(memory_space=pltpu.VMEM),
        scratch_shapes=[
            pltpu.VMEM((m_per, k), jnp.float32),
            pltpu.VMEM((k, n_per), jnp.float32),
            pltpu.VMEM((k, n_per), jnp.float32),
            pltpu.VMEM((m_per, n), jnp.float32),
        ] + [
            pltpu.VMEM((m_per, n_per), F8)
            for _ in range(N_DEV)
        ] + [
            pltpu.VMEM((N_DEV, m_per, n_per), F8),
            pltpu.VMEM((N_DEV, 128), jnp.float32),
            pltpu.SemaphoreType.DMA,
            pltpu.SemaphoreType.DMA((2,)),
            pltpu.SemaphoreType.DMA((N_DEV,)),
            pltpu.SemaphoreType.DMA((N_DEV,)),
            pltpu.SemaphoreType.DMA((N_DEV,)),
            pltpu.SemaphoreType.DMA((N_DEV,)),
        ],
        compiler_params=pltpu.CompilerParams(
            collective_id=0,
            vmem_limit_bytes=100 * 1024 * 1024,
        ),
    )(x, w_mat)


# device time: 41117 ns/iter; 1.1152x vs baseline; 1.0866x over previous
import jax
import jax.numpy as jnp
from jax import lax
from jax.experimental import pallas as pl
from jax.experimental.pallas import tpu as pltpu

N_DEV = 8
F8 = jnp.float8_e4m3fn


def kernel(x, w_mat):
    m_per, k = x.shape
    _, n = w_mat.shape
    n_per = n // N_DEV

    def body(x_ref, w_ref, out_ref, y_ref, qs_ref, qr_ref, gmax_ref,
             amax_ref, amax_send_sems, amax_recv_sems, send_sems, recv_sems):
        j = pl.program_id(0)
        me = lax.axis_index("i")

        @pl.when(j < N_DEV)
        def _():
            blk = jnp.maximum(
                jnp.dot(x_ref[...], w_ref[...],
                        preferred_element_type=jnp.float32),
                0.0,
            )
            y_ref[:, pl.ds(j * n_per, n_per)] = blk
            bmax = jnp.full((1, 128), jnp.max(blk), jnp.float32)

            @pl.when(j == 0)
            def _():
                amax_ref[...] = bmax

            @pl.when(j > 0)
            def _():
                amax_ref[...] = jnp.maximum(amax_ref[...], bmax)

        @pl.when(j == N_DEV)
        def _():
            barrier = pltpu.get_barrier_semaphore()
            for d in range(N_DEV):
                @pl.when(me != d)
                def _():
                    pl.semaphore_signal(
                        barrier, inc=1,
                        device_id=(d,), device_id_type=pl.DeviceIdType.MESH,
                    )
            pl.semaphore_wait(barrier, N_DEV - 1)

            gmax_ref[pl.ds(me, 1), :] = amax_ref[...]
            for o in range(1, N_DEV):
                dst = (me + o) % N_DEV
                pltpu.make_async_remote_copy(
                    src_ref=gmax_ref.at[pl.ds(me, 1), :],
                    dst_ref=gmax_ref.at[pl.ds(me, 1), :],
                    send_sem=amax_send_sems.at[dst],
                    recv_sem=amax_recv_sems.at[me],
                    device_id=(dst,),
                    device_id_type=pl.DeviceIdType.MESH,
                ).start()
            for t in range(N_DEV):
                @pl.when(me != t)
                def _(t=t):
                    pltpu.make_async_remote_copy(
                        src_ref=gmax_ref.at[pl.ds(t, 1), :],
                        dst_ref=gmax_ref.at[pl.ds(t, 1), :],
                        send_sem=amax_send_sems.at[t],
                        recv_sem=amax_recv_sems.at[t],
                        device_id=(t,),
                        device_id_type=pl.DeviceIdType.MESH,
                    ).wait_recv()

            gmax = jnp.max(gmax_ref[...])
            scale = gmax / 448.0
            inv_scale = 1.0 / scale

            for b in range(N_DEV):
                qs_ref[b, :, :] = (
                    y_ref[:, b * n_per:(b + 1) * n_per] * inv_scale
                ).astype(F8)

                @pl.when(me != b)
                def _(b=b):
                    pltpu.make_async_remote_copy(
                        src_ref=qs_ref.at[b],
                        dst_ref=qr_ref.at[me],
                        send_sem=send_sems.at[b],
                        recv_sem=recv_sems.at[me],
                        device_id=(b,),
                        device_id_type=pl.DeviceIdType.MESH,
                    ).start()

            own = qs_ref[pl.ds(me, 1)]
            out_ref[pl.ds(me * m_per, m_per), :] = (
                own[0].astype(jnp.float32) * scale)

            for t in range(N_DEV):
                @pl.when(me != t)
                def _(t=t):
                    pltpu.make_async_remote_copy(
                        src_ref=qs_ref.at[t],
                        dst_ref=qr_ref.at[t],
                        send_sem=send_sems.at[t],
                        recv_sem=recv_sems.at[t],
                        device_id=(t,),
                        device_id_type=pl.DeviceIdType.MESH,
                    ).wait_recv()
                    out_ref[t * m_per:(t + 1) * m_per, :] = (
                        qr_ref[t].astype(jnp.float32) * scale
                    )

            for t in range(N_DEV):
                @pl.when(me != t)
                def _(t=t):
                    pltpu.make_async_remote_copy(
                        src_ref=qs_ref.at[t],
                        dst_ref=qr_ref.at[t],
                        send_sem=send_sems.at[t],
                        recv_sem=recv_sems.at[t],
                        device_id=(t,),
                        device_id_type=pl.DeviceIdType.MESH,
                    ).wait_send()
                    pltpu.make_async_remote_copy(
                        src_ref=gmax_ref.at[pl.ds(t, 1), :],
                        dst_ref=gmax_ref.at[pl.ds(t, 1), :],
                        send_sem=amax_send_sems.at[t],
                        recv_sem=amax_recv_sems.at[t],
                        device_id=(t,),
                        device_id_type=pl.DeviceIdType.MESH,
                    ).wait_send()

    out_shape = jax.ShapeDtypeStruct((N_DEV * m_per, n_per), jnp.float32)
    return pl.pallas_call(
        body,
        grid=(N_DEV + 1,),
        in_specs=[
            pl.BlockSpec((m_per, k), lambda j: (0, 0)),
            pl.BlockSpec((k, n_per), lambda j: (0, jnp.minimum(j, N_DEV - 1))),
        ],
        out_specs=pl.BlockSpec((N_DEV * m_per, n_per), lambda j: (0, 0)),
        out_shape=out_shape,
        scratch_shapes=[
            pltpu.VMEM((m_per, n), jnp.float32),
            pltpu.VMEM((N_DEV, m_per, n_per), F8),
            pltpu.VMEM((N_DEV, m_per, n_per), F8),
            pltpu.VMEM((N_DEV, 128), jnp.float32),
            pltpu.VMEM((1, 128), jnp.float32),
            pltpu.SemaphoreType.DMA((N_DEV,)),
            pltpu.SemaphoreType.DMA((N_DEV,)),
            pltpu.SemaphoreType.DMA((N_DEV,)),
            pltpu.SemaphoreType.DMA((N_DEV,)),
        ],
        compiler_params=pltpu.CompilerParams(
            collective_id=0,
            dimension_semantics=("arbitrary",),
            vmem_limit_bytes=100 * 1024 * 1024,
        ),
    )(x, w_mat)


# device time: 40830 ns/iter; 1.1230x vs baseline; 1.0070x over previous
import jax
import jax.numpy as jnp
from jax import lax
from jax.experimental import pallas as pl
from jax.experimental.pallas import tpu as pltpu

N_DEV = 8
F8 = jnp.float8_e4m3fn


def kernel(x, w_mat):
    m_per, k = x.shape
    _, n = w_mat.shape
    n_per = n // N_DEV

    def body(x_ref, w_ref, out_ref, y_ref, qs_ref, qr_ref, rds_ref, rdr_ref,
             amax_ref, rd_send_sems, rd_recv_sems, send_sems, recv_sems):
        j = pl.program_id(0)
        me = lax.axis_index("i")

        @pl.when(j == 0)
        def _():
            barrier = pltpu.get_barrier_semaphore()
            for d in range(N_DEV):
                @pl.when(me != d)
                def _():
                    pl.semaphore_signal(
                        barrier, inc=1,
                        device_id=(d,), device_id_type=pl.DeviceIdType.MESH,
                    )

        @pl.when(j < N_DEV)
        def _():
            blk = jnp.maximum(
                jnp.dot(x_ref[...], w_ref[...],
                        preferred_element_type=jnp.float32),
                0.0,
            )
            y_ref[:, pl.ds(j * n_per, n_per)] = blk
            bmax = jnp.full((1, 128), jnp.max(blk), jnp.float32)

            @pl.when(j == 0)
            def _():
                amax_ref[...] = bmax

            @pl.when(j > 0)
            def _():
                amax_ref[...] = jnp.maximum(amax_ref[...], bmax)

        @pl.when(j == N_DEV)
        def _():
            barrier = pltpu.get_barrier_semaphore()
            pl.semaphore_wait(barrier, N_DEV - 1)

            running = amax_ref[...]
            for r in range(3):
                partner = jnp.bitwise_xor(me, 1 << r)
                rds_ref[pl.ds(r, 1), :] = running
                pltpu.make_async_remote_copy(
                    src_ref=rds_ref.at[pl.ds(r, 1), :],
                    dst_ref=rdr_ref.at[pl.ds(r, 1), :],
                    send_sem=rd_send_sems.at[r],
                    recv_sem=rd_recv_sems.at[r],
                    device_id=(partner,),
                    device_id_type=pl.DeviceIdType.MESH,
                ).start()
                pltpu.make_async_remote_copy(
                    src_ref=rds_ref.at[pl.ds(r, 1), :],
                    dst_ref=rdr_ref.at[pl.ds(r, 1), :],
                    send_sem=rd_send_sems.at[r],
                    recv_sem=rd_recv_sems.at[r],
                    device_id=(partner,),
                    device_id_type=pl.DeviceIdType.MESH,
                ).wait_recv()
                running = jnp.maximum(running, rdr_ref[pl.ds(r, 1), :])

            gmax = jnp.max(running)
            scale = gmax / 448.0
            inv_scale = 1.0 / scale

            for b in range(N_DEV):
                qs_ref[b, :, :] = (
                    y_ref[:, b * n_per:(b + 1) * n_per] * inv_scale
                ).astype(F8)

                @pl.when(me != b)
                def _(b=b):
                    pltpu.make_async_remote_copy(
                        src_ref=qs_ref.at[b],
                        dst_ref=qr_ref.at[me],
                        send_sem=send_sems.at[b],
                        recv_sem=recv_sems.at[me],
                        device_id=(b,),
                        device_id_type=pl.DeviceIdType.MESH,
                    ).start()

            own = qs_ref[pl.ds(me, 1)]
            out_ref[pl.ds(me * m_per, m_per), :] = (
                own[0].astype(jnp.float32) * scale)

            for t in range(N_DEV):
                @pl.when(me != t)
                def _(t=t):
                    pltpu.make_async_remote_copy(
                        src_ref=qs_ref.at[t],
                        dst_ref=qr_ref.at[t],
                        send_sem=send_sems.at[t],
                        recv_sem=recv_sems.at[t],
                        device_id=(t,),
                        device_id_type=pl.DeviceIdType.MESH,
                    ).wait_recv()
                    out_ref[t * m_per:(t + 1) * m_per, :] = (
                        qr_ref[t].astype(jnp.float32) * scale
                    )

            for t in range(N_DEV):
                @pl.when(me != t)
                def _(t=t):
                    pltpu.make_async_remote_copy(
                        src_ref=qs_ref.at[t],
                        dst_ref=qr_ref.at[t],
                        send_sem=send_sems.at[t],
                        recv_sem=recv_sems.at[t],
                        device_id=(t,),
                        device_id_type=pl.DeviceIdType.MESH,
                    ).wait_send()
            for r in range(3):
                pltpu.make_async_remote_copy(
                    src_ref=rds_ref.at[pl.ds(r, 1), :],
                    dst_ref=rdr_ref.at[pl.ds(r, 1), :],
                    send_sem=rd_send_sems.at[r],
                    recv_sem=rd_recv_sems.at[r],
                    device_id=(jnp.bitwise_xor(me, 1 << r),),
                    device_id_type=pl.DeviceIdType.MESH,
                ).wait_send()

    out_shape = jax.ShapeDtypeStruct((N_DEV * m_per, n_per), jnp.float32)
    return pl.pallas_call(
        body,
        grid=(N_DEV + 1,),
        in_specs=[
            pl.BlockSpec((m_per, k), lambda j: (0, 0)),
            pl.BlockSpec((k, n_per), lambda j: (0, jnp.minimum(j, N_DEV - 1))),
        ],
        out_specs=pl.BlockSpec((N_DEV * m_per, n_per), lambda j: (0, 0)),
        out_shape=out_shape,
        scratch_shapes=[
            pltpu.VMEM((m_per, n), jnp.float32),
            pltpu.VMEM((N_DEV, m_per, n_per), F8),
            pltpu.VMEM((N_DEV, m_per, n_per), F8),
            pltpu.VMEM((3, 128), jnp.float32),
            pltpu.VMEM((3, 128), jnp.float32),
            pltpu.VMEM((1, 128), jnp.float32),
            pltpu.SemaphoreType.DMA((3,)),
            pltpu.SemaphoreType.DMA((3,)),
            pltpu.SemaphoreType.DMA((N_DEV,)),
            pltpu.SemaphoreType.DMA((N_DEV,)),
        ],
        compiler_params=pltpu.CompilerParams(
            collective_id=0,
            dimension_semantics=("arbitrary",),
            vmem_limit_bytes=100 * 1024 * 1024,
        ),
    )(x, w_mat)


# device time: 39678 ns/iter; 1.1556x vs baseline; 1.0290x over previous
import jax
import jax.numpy as jnp
from jax import lax
from jax.experimental import pallas as pl
from jax.experimental.pallas import tpu as pltpu

N_DEV = 8
F8 = jnp.float8_e4m3fn


def kernel(x, w_mat):
    m_per, k = x.shape
    _, n = w_mat.shape
    n_per = n // N_DEV

    def body(x_ref, w_ref, out_ref, y_ref, qs_ref, qr_ref, ostage_ref,
             rds_ref, rdr_ref, amax_ref, rd_send_sems, rd_recv_sems,
             send_sems, recv_sems, out_sems):
        j = pl.program_id(0)
        me = lax.axis_index("i")

        @pl.when(j == 0)
        def _():
            barrier = pltpu.get_barrier_semaphore()
            for d in range(N_DEV):
                @pl.when(me != d)
                def _():
                    pl.semaphore_signal(
                        barrier, inc=1,
                        device_id=(d,), device_id_type=pl.DeviceIdType.MESH,
                    )

        @pl.when(j < N_DEV)
        def _():
            blk = jnp.maximum(
                jnp.dot(x_ref[...], w_ref[...],
                        preferred_element_type=jnp.float32),
                0.0,
            )
            y_ref[:, pl.ds(j * n_per, n_per)] = blk
            bmax = jnp.full((1, 128), jnp.max(blk), jnp.float32)

            @pl.when(j == 0)
            def _():
                amax_ref[...] = bmax

            @pl.when(j > 0)
            def _():
                amax_ref[...] = jnp.maximum(amax_ref[...], bmax)

        @pl.when(j == N_DEV)
        def _():
            barrier = pltpu.get_barrier_semaphore()
            pl.semaphore_wait(barrier, N_DEV - 1)

            running = amax_ref[...]
            for r in range(3):
                partner = jnp.bitwise_xor(me, 1 << r)
                rds_ref[pl.ds(r, 1), :] = running
                pltpu.make_async_remote_copy(
                    src_ref=rds_ref.at[pl.ds(r, 1), :],
                    dst_ref=rdr_ref.at[pl.ds(r, 1), :],
                    send_sem=rd_send_sems.at[r],
                    recv_sem=rd_recv_sems.at[r],
                    device_id=(partner,),
                    device_id_type=pl.DeviceIdType.MESH,
                ).start()
                pltpu.make_async_remote_copy(
                    src_ref=rds_ref.at[pl.ds(r, 1), :],
                    dst_ref=rdr_ref.at[pl.ds(r, 1), :],
                    send_sem=rd_send_sems.at[r],
                    recv_sem=rd_recv_sems.at[r],
                    device_id=(partner,),
                    device_id_type=pl.DeviceIdType.MESH,
                ).wait_recv()
                running = jnp.maximum(running, rdr_ref[pl.ds(r, 1), :])

            gmax = jnp.max(running)
            scale = gmax / 448.0
            inv_scale = 1.0 / scale

            for b in range(N_DEV):
                qs_ref[b, :, :] = (
                    y_ref[:, b * n_per:(b + 1) * n_per] * inv_scale
                ).astype(F8)

                @pl.when(me != b)
                def _(b=b):
                    pltpu.make_async_remote_copy(
                        src_ref=qs_ref.at[b],
                        dst_ref=qr_ref.at[me],
                        send_sem=send_sems.at[b],
                        recv_sem=recv_sems.at[me],
                        device_id=(b,),
                        device_id_type=pl.DeviceIdType.MESH,
                    ).start()

            own = qs_ref[pl.ds(me, 1)]
            ostage_ref[pl.ds(me, 1)] = own.astype(jnp.float32) * scale
            pltpu.make_async_copy(
                ostage_ref.at[me],
                out_ref.at[pl.ds(me * m_per, m_per), :],
                out_sems.at[me],
            ).start()

            for t in range(N_DEV):
                @pl.when(me != t)
                def _(t=t):
                    pltpu.make_async_remote_copy(
                        src_ref=qs_ref.at[t],
                        dst_ref=qr_ref.at[t],
                        send_sem=send_sems.at[t],
                        recv_sem=recv_sems.at[t],
                        device_id=(t,),
                        device_id_type=pl.DeviceIdType.MESH,
                    ).wait_recv()
                    ostage_ref[t, :, :] = qr_ref[t].astype(jnp.float32) * scale
                    pltpu.make_async_copy(
                        ostage_ref.at[t],
                        out_ref.at[t * m_per:(t + 1) * m_per, :],
                        out_sems.at[t],
                    ).start()

            for t in range(N_DEV):
                @pl.when(me != t)
                def _(t=t):
                    pltpu.make_async_remote_copy(
                        src_ref=qs_ref.at[t],
                        dst_ref=qr_ref.at[t],
                        send_sem=send_sems.at[t],
                        recv_sem=recv_sems.at[t],
                        device_id=(t,),
                        device_id_type=pl.DeviceIdType.MESH,
                    ).wait_send()
            for r in range(3):
                pltpu.make_async_remote_copy(
                    src_ref=rds_ref.at[pl.ds(r, 1), :],
                    dst_ref=rdr_ref.at[pl.ds(r, 1), :],
                    send_sem=rd_send_sems.at[r],
                    recv_sem=rd_recv_sems.at[r],
                    device_id=(jnp.bitwise_xor(me, 1 << r),),
                    device_id_type=pl.DeviceIdType.MESH,
                ).wait_send()

            for t in range(N_DEV):
                pltpu.make_async_copy(
                    ostage_ref.at[t],
                    out_ref.at[t * m_per:(t + 1) * m_per, :],
                    out_sems.at[t],
                ).wait()

    out_shape = jax.ShapeDtypeStruct((N_DEV * m_per, n_per), jnp.float32)
    return pl.pallas_call(
        body,
        grid=(N_DEV + 1,),
        in_specs=[
            pl.BlockSpec((m_per, k), lambda j: (0, 0)),
            pl.BlockSpec((k, n_per), lambda j: (0, jnp.minimum(j, N_DEV - 1))),
        ],
        out_specs=pl.BlockSpec(memory_space=pl.ANY),
        out_shape=out_shape,
        scratch_shapes=[
            pltpu.VMEM((m_per, n), jnp.float32),
            pltpu.VMEM((N_DEV, m_per, n_per), F8),
            pltpu.VMEM((N_DEV, m_per, n_per), F8),
            pltpu.VMEM((N_DEV, m_per, n_per), jnp.float32),
            pltpu.VMEM((3, 128), jnp.float32),
            pltpu.VMEM((3, 128), jnp.float32),
            pltpu.VMEM((1, 128), jnp.float32),
            pltpu.SemaphoreType.DMA((3,)),
            pltpu.SemaphoreType.DMA((3,)),
            pltpu.SemaphoreType.DMA((N_DEV,)),
            pltpu.SemaphoreType.DMA((N_DEV,)),
            pltpu.SemaphoreType.DMA((N_DEV,)),
        ],
        compiler_params=pltpu.CompilerParams(
            collective_id=0,
            dimension_semantics=("arbitrary",),
            vmem_limit_bytes=100 * 1024 * 1024,
        ),
    )(x, w_mat)
